# baseline (speedup 1.0000x reference)
import jax
import jax.numpy as jnp
from jax.experimental import pallas as pl
from jax.experimental.pallas import tpu as pltpu

_LANES = 128
_SEG_LANE = 100


def _edge_kernel(ei_ref, rel_ref, x_ref, re_ref, w_ref, acc_ref):
    p = pl.program_id(0)
    b = pl.program_id(1)
    blk = ei_ref.shape[1]
    mask_seg = (jax.lax.broadcasted_iota(jnp.int32, (1, _LANES), 1)
                == _SEG_LANE).astype(jnp.float32)
    w1 = w_ref[0:1, :]
    w2 = w_ref[1:2, :]
    w3 = w_ref[2:3, :]

    @pl.when((p == 0) & (b == 0))
    def _init():
        acc_ref[...] = jnp.zeros_like(acc_ref)

    def _edge(e):
        i = ei_ref[0, e]
        j = ei_ref[1, e]
        r = rel_ref[0, e]
        xi = x_ref[pl.ds(i, 1), :]
        xj = x_ref[pl.ds(j, 1), :]
        rr = re_ref[pl.ds(r, 1), :]
        v = xi * w1 + rr * w2 + xj * w3
        att_raw = jnp.sum(v, axis=1, keepdims=True)
        att_exp = jnp.exp(att_raw)
        return i, xi, xj, rr, att_exp

    @pl.when(p == 0)
    def _pass1():
        def body(e, carry):
            i, _, _, _, att_exp = _edge(e)
            row = acc_ref[pl.ds(i, 1), :]
            acc_ref[pl.ds(i, 1), :] = row + att_exp * mask_seg
            return carry
        jax.lax.fori_loop(0, blk, body, 0, unroll=False)

    @pl.when(p == 1)
    def _pass2():
        def body(e, carry):
            i, xi, xj, rr, att_exp = _edge(e)
            row = acc_ref[pl.ds(i, 1), :]
            seg = jnp.sum(row * mask_seg, axis=1, keepdims=True)
            att = att_exp / (seg + 1e-16)
            msg = (xi + xj + rr) * (att * (1.0 / 3.0))
            acc_ref[pl.ds(i, 1), :] = row + msg
            return carry
        jax.lax.fori_loop(0, blk, body, 0, unroll=False)


def _combine_kernel(x_ref, acc_ref, o_ref):
    o_ref[...] = jnp.maximum(x_ref[...] + acc_ref[...], 0.0)


def kernel(x, edge_index_all, rel_all, rel_emb, W):
    n, d = x.shape
    e = edge_index_all.shape[1]
    r = rel_emb.shape[0]
    xp = jnp.pad(x, ((0, 0), (0, _LANES - d)))
    rep = jnp.pad(rel_emb, ((0, 0), (0, _LANES - d)))
    wp = jnp.pad(W.reshape(3, d), ((0, 0), (0, _LANES - d)))

    blk = 3200 if e % 3200 == 0 else e
    nblk = e // blk
    rel2 = rel_all.reshape(1, e)

    acc = pl.pallas_call(
        _edge_kernel,
        grid=(2, nblk),
        in_specs=[
            pl.BlockSpec((2, blk), lambda p, b: (0, b), memory_space=pltpu.SMEM),
            pl.BlockSpec((1, blk), lambda p, b: (0, b), memory_space=pltpu.SMEM),
            pl.BlockSpec((n, _LANES), lambda p, b: (0, 0)),
            pl.BlockSpec((r, _LANES), lambda p, b: (0, 0)),
            pl.BlockSpec((3, _LANES), lambda p, b: (0, 0)),
        ],
        out_specs=pl.BlockSpec((n, _LANES), lambda p, b: (0, 0)),
        out_shape=jax.ShapeDtypeStruct((n, _LANES), jnp.float32),
    )(edge_index_all, rel2, xp, rep, wp)

    rb = 1000 if n % 1000 == 0 else n
    out = pl.pallas_call(
        _combine_kernel,
        grid=(n // rb,),
        in_specs=[
            pl.BlockSpec((rb, d), lambda b: (b, 0)),
            pl.BlockSpec((rb, d), lambda b: (b, 0)),
        ],
        out_specs=pl.BlockSpec((rb, d), lambda b: (b, 0)),
        out_shape=jax.ShapeDtypeStruct((n, d), jnp.float32),
    )(x, acc[:, :d])
    return out


# speedup vs baseline: 1.7725x; 1.7725x over previous
import jax
import jax.numpy as jnp
from jax.experimental import pallas as pl
from jax.experimental.pallas import tpu as pltpu

_LANES = 128
_SEG_LANE = 100


def _edge_kernel(ei_ref, rel_ref, x_ref, re_ref, w_ref, acc_ref):
    p = pl.program_id(0)
    b = pl.program_id(1)
    blk = ei_ref.shape[1]
    mask_seg = (jax.lax.broadcasted_iota(jnp.int32, (1, _LANES), 1)
                == _SEG_LANE).astype(jnp.float32)
    w1 = w_ref[0:1, :]
    w2 = w_ref[1:2, :]
    w3 = w_ref[2:3, :]

    @pl.when((p == 0) & (b == 0))
    def _init():
        acc_ref[...] = jnp.zeros_like(acc_ref)

    def _edge(e):
        i = ei_ref[0, e]
        j = ei_ref[1, e]
        r = rel_ref[0, e]
        xi = x_ref[pl.ds(i, 1), :]
        xj = x_ref[pl.ds(j, 1), :]
        rr = re_ref[pl.ds(r, 1), :]
        v = xi * w1 + rr * w2 + xj * w3
        att_raw = jnp.sum(v, axis=1, keepdims=True)
        att_exp = jnp.exp(att_raw)
        return i, xi, xj, rr, att_exp

    @pl.when(p == 0)
    def _pass1():
        def body(e, carry):
            i, _, _, _, att_exp = _edge(e)
            row = acc_ref[pl.ds(i, 1), :]
            acc_ref[pl.ds(i, 1), :] = row + att_exp * mask_seg
            return carry
        jax.lax.fori_loop(0, blk, body, 0, unroll=8)

    @pl.when(p == 1)
    def _pass2():
        def body(e, carry):
            i, xi, xj, rr, att_exp = _edge(e)
            row = acc_ref[pl.ds(i, 1), :]
            seg = jnp.sum(row * mask_seg, axis=1, keepdims=True)
            att = att_exp / (seg + 1e-16)
            msg = (xi + xj + rr) * (att * (1.0 / 3.0))
            acc_ref[pl.ds(i, 1), :] = row + msg
            return carry
        jax.lax.fori_loop(0, blk, body, 0, unroll=8)


def _combine_kernel(x_ref, acc_ref, o_ref):
    o_ref[...] = jnp.maximum(x_ref[...] + acc_ref[...], 0.0)


def kernel(x, edge_index_all, rel_all, rel_emb, W):
    n, d = x.shape
    e = edge_index_all.shape[1]
    r = rel_emb.shape[0]
    xp = jnp.pad(x, ((0, 0), (0, _LANES - d)))
    rep = jnp.pad(rel_emb, ((0, 0), (0, _LANES - d)))
    wp = jnp.pad(W.reshape(3, d), ((0, 0), (0, _LANES - d)))

    blk = 3200 if e % 3200 == 0 else e
    nblk = e // blk
    rel2 = rel_all.reshape(1, e)

    acc = pl.pallas_call(
        _edge_kernel,
        grid=(2, nblk),
        in_specs=[
            pl.BlockSpec((2, blk), lambda p, b: (0, b), memory_space=pltpu.SMEM),
            pl.BlockSpec((1, blk), lambda p, b: (0, b), memory_space=pltpu.SMEM),
            pl.BlockSpec((n, _LANES), lambda p, b: (0, 0)),
            pl.BlockSpec((r, _LANES), lambda p, b: (0, 0)),
            pl.BlockSpec((3, _LANES), lambda p, b: (0, 0)),
        ],
        out_specs=pl.BlockSpec((n, _LANES), lambda p, b: (0, 0)),
        out_shape=jax.ShapeDtypeStruct((n, _LANES), jnp.float32),
    )(edge_index_all, rel2, xp, rep, wp)

    rb = 1000 if n % 1000 == 0 else n
    out = pl.pallas_call(
        _combine_kernel,
        grid=(n // rb,),
        in_specs=[
            pl.BlockSpec((rb, d), lambda b: (b, 0)),
            pl.BlockSpec((rb, d), lambda b: (b, 0)),
        ],
        out_specs=pl.BlockSpec((rb, d), lambda b: (b, 0)),
        out_shape=jax.ShapeDtypeStruct((n, d), jnp.float32),
    )(x, acc[:, :d])
    return out
